# baseline (speedup 1.0000x reference)
import functools

import jax
import jax.numpy as jnp
from jax import lax
from jax.experimental import pallas as pl
from jax.experimental.pallas import tpu as pltpu
from jax.experimental.pallas import tpu_sc as plsc

BATCH = 4
SEQ_LEN = 2048
N_EMBED = 128

_NUM_CORES = 2
_NUM_SUBCORES = 16
_NW = _NUM_CORES * _NUM_SUBCORES
_SPW = SEQ_LEN // _NW
_NH = 2
_HPW = _SPW // _NH
_LANES = 16
_VPR = N_EMBED // _LANES


def _emb_body(idx_hbm, tok_hbm, pos_hbm, out_hbm, idx_v, rows_v, pos_v,
              gsems, psem):
    wid = lax.axis_index("s") * _NUM_CORES + lax.axis_index("c")
    s_start = wid * _SPW

    idx_cps = [
        pltpu.async_copy(idx_hbm.at[b, pl.ds(s_start, _SPW)], idx_v.at[b],
                         psem)
        for b in range(BATCH)
    ]
    pos_cp = pltpu.async_copy(pos_hbm.at[pl.ds(s_start, _SPW)], pos_v, psem)
    for cp in idx_cps:
        cp.wait()

    gathers = [
        [
            pltpu.async_copy(
                tok_hbm.at[idx_v.at[b, pl.ds(h * _HPW, _HPW)]],
                rows_v.at[b, pl.ds(h * _HPW, _HPW), :],
                gsems.at[h * BATCH + b],
            )
            for b in range(BATCH)
        ]
        for h in range(_NH)
    ]
    pos_cp.wait()

    stores = []
    for h in range(_NH):
        for cp in gathers[h]:
            cp.wait()

        def add_row(r, _, h=h):
            row = h * _HPW + r
            for c in range(_VPR):
                sl = pl.ds(c * _LANES, _LANES)
                p = pos_v[row, sl]
                for b in range(BATCH):
                    rows_v[b, row, sl] = rows_v[b, row, sl] + p
            return 0

        lax.fori_loop(0, _HPW, add_row, 0)
        stores.extend(
            pltpu.async_copy(
                rows_v.at[b, pl.ds(h * _HPW, _HPW), :],
                out_hbm.at[b, pl.ds(s_start + h * _HPW, _HPW), :],
                psem,
            )
            for b in range(BATCH)
        )
    for cp in stores:
        cp.wait()


@jax.jit
def kernel(input_ids, token_table, pos_table):
    idx = input_ids.astype(jnp.int32)

    mesh = plsc.VectorSubcoreMesh(core_axis_name="c", subcore_axis_name="s")
    emb = functools.partial(
        pl.kernel,
        mesh=mesh,
        out_type=jax.ShapeDtypeStruct((BATCH, SEQ_LEN, N_EMBED), jnp.float32),
        scratch_types=[
            pltpu.VMEM((BATCH, _SPW), jnp.int32),
            pltpu.VMEM((BATCH, _SPW, N_EMBED), jnp.float32),
            pltpu.VMEM((_SPW, N_EMBED), jnp.float32),
            pltpu.SemaphoreType.DMA((_NH * BATCH,)),
            pltpu.SemaphoreType.DMA,
        ],
    )(_emb_body)

    return emb(idx, token_table, pos_table)
